# baseline (speedup 1.0000x reference)
import functools

import jax
import jax.numpy as jnp
from jax.experimental import pallas as pl
from jax.experimental.pallas import tpu as pltpu


def _body(t_ref, x_ref, w1_ref, b1_ref, w2_ref, b2_ref, wp_ref, g1_ref,
          c_ref, o_ref, *, n_types, out_d, hidden, n_split):
    bm = x_ref.shape[0]
    hm = bm // n_split
    for s in range(n_split):
        rows = slice(s * hm, (s + 1) * hm)
        x = x_ref[rows, :].astype(jnp.bfloat16)
        h = jnp.dot(x, w1_ref[...], preferred_element_type=jnp.float32)
        h = jnp.maximum(h + b1_ref[...], 0.0)
        h = jnp.dot(h.astype(jnp.bfloat16), w2_ref[...],
                    preferred_element_type=jnp.float32)
        h = h + b2_ref[...]
        y = jnp.dot(h.astype(jnp.bfloat16), wp_ref[...],
                    preferred_element_type=jnp.float32)
        inv_h = 1.0 / hidden
        mu = jnp.sum(h, axis=-1, keepdims=True) * inv_h
        m2 = jnp.sum(h * h, axis=-1, keepdims=True) * inv_h
        rstd = jax.lax.rsqrt(jnp.maximum(m2 - mu * mu, 0.0) + 1e-5)
        t = t_ref[rows, :]
        y_sel = None
        g_sel = None
        c_sel = None
        for tt in range(n_types):
            mask = t == tt
            sl = slice(tt * out_d, (tt + 1) * out_d)
            ys = jnp.where(mask, y[:, sl], 0.0)
            gs = jnp.where(mask, g1_ref[:, sl], 0.0)
            cs = jnp.where(mask, c_ref[:, sl], 0.0)
            if y_sel is None:
                y_sel, g_sel, c_sel = ys, gs, cs
            else:
                y_sel, g_sel, c_sel = y_sel + ys, g_sel + gs, c_sel + cs
        o_ref[rows, :] = rstd * y_sel - (rstd * mu) * g_sel + c_sel


def kernel(node_latent, node_types, w1, b1, w2, b2, ln_gamma, ln_beta,
           head_w, head_b, *, interpret=False):
    n, latent = node_latent.shape
    hidden = w1.shape[1]
    out_d = head_w.shape[2]
    n_types = head_w.shape[0]
    bm = 1000
    grid = (n // bm,)

    t2 = node_types.reshape(n, 1)
    b1r = b1.reshape(1, hidden)
    b2r = b2.reshape(1, hidden)
    w1b = w1.astype(jnp.bfloat16)
    w2b = w2.astype(jnp.bfloat16)
    w_cat = head_w.transpose(1, 0, 2).reshape(hidden, n_types * out_d)
    wp = (ln_gamma[:, None] * w_cat).astype(jnp.bfloat16)
    g1 = (ln_gamma @ w_cat).reshape(1, n_types * out_d)
    c_all = (ln_beta @ w_cat).reshape(1, n_types * out_d) \
        + head_b.reshape(1, n_types * out_d)

    return pl.pallas_call(
        functools.partial(_body, n_types=n_types, out_d=out_d, hidden=hidden,
                          n_split=2),
        out_shape=jax.ShapeDtypeStruct((n, out_d), jnp.float32),
        grid=grid,
        in_specs=[
            pl.BlockSpec((bm, 1), lambda i: (i, 0)),
            pl.BlockSpec((bm, latent), lambda i: (i, 0)),
            pl.BlockSpec((latent, hidden), lambda i: (0, 0)),
            pl.BlockSpec((1, hidden), lambda i: (0, 0)),
            pl.BlockSpec((hidden, hidden), lambda i: (0, 0)),
            pl.BlockSpec((1, hidden), lambda i: (0, 0)),
            pl.BlockSpec((hidden, n_types * out_d), lambda i: (0, 0)),
            pl.BlockSpec((1, n_types * out_d), lambda i: (0, 0)),
            pl.BlockSpec((1, n_types * out_d), lambda i: (0, 0)),
        ],
        out_specs=pl.BlockSpec((bm, out_d), lambda i: (i, 0)),
        compiler_params=pltpu.CompilerParams(
            dimension_semantics=("parallel",),
            vmem_limit_bytes=56 * 1024 * 1024,
        ),
        name="adaptive_decoder",
        interpret=interpret,
    )(t2, node_latent, w1b, b1r, w2b, b2r, wp, g1, c_all)


# speedup vs baseline: 1.0307x; 1.0307x over previous
import functools

import jax
import jax.numpy as jnp
from jax.experimental import pallas as pl
from jax.experimental.pallas import tpu as pltpu


def _body(t_ref, x_ref, w1_ref, b1_ref, w2_ref, b2_ref, wp_ref, g1_ref,
          c_ref, o_ref, *, n_types, out_d, hidden, n_split):
    bm = x_ref.shape[0]
    hm = bm // n_split
    for s in range(n_split):
        rows = slice(s * hm, (s + 1) * hm)
        x = x_ref[rows, :].astype(jnp.bfloat16)
        h = jnp.dot(x, w1_ref[...], preferred_element_type=jnp.float32)
        h = jnp.maximum(h + b1_ref[...], 0.0)
        h = jnp.dot(h.astype(jnp.bfloat16), w2_ref[...],
                    preferred_element_type=jnp.float32)
        h = h + b2_ref[...]
        y = jnp.dot(h.astype(jnp.bfloat16), wp_ref[...],
                    preferred_element_type=jnp.float32)
        inv_h = 1.0 / hidden
        mu = jnp.sum(h, axis=-1, keepdims=True) * inv_h
        m2 = jnp.sum(h * h, axis=-1, keepdims=True) * inv_h
        rstd = jax.lax.rsqrt(jnp.maximum(m2 - mu * mu, 0.0) + 1e-5)
        t = t_ref[rows, :]
        y_sel = None
        g_sel = None
        c_sel = None
        for tt in range(n_types):
            mask = t == tt
            sl = slice(tt * out_d, (tt + 1) * out_d)
            ys = jnp.where(mask, y[:, sl], 0.0)
            gs = jnp.where(mask, g1_ref[:, sl], 0.0)
            cs = jnp.where(mask, c_ref[:, sl], 0.0)
            if y_sel is None:
                y_sel, g_sel, c_sel = ys, gs, cs
            else:
                y_sel, g_sel, c_sel = y_sel + ys, g_sel + gs, c_sel + cs
        o_ref[rows, :] = rstd * y_sel - (rstd * mu) * g_sel + c_sel


def kernel(node_latent, node_types, w1, b1, w2, b2, ln_gamma, ln_beta,
           head_w, head_b, *, interpret=False):
    n, latent = node_latent.shape
    hidden = w1.shape[1]
    out_d = head_w.shape[2]
    n_types = head_w.shape[0]
    bm = 1000
    grid = (n // bm,)

    t2 = node_types.reshape(n, 1)
    b1r = b1.reshape(1, hidden)
    b2r = b2.reshape(1, hidden)
    w1b = w1.astype(jnp.bfloat16)
    w2b = w2.astype(jnp.bfloat16)
    w_cat = head_w.transpose(1, 0, 2).reshape(hidden, n_types * out_d)
    wp = (ln_gamma[:, None] * w_cat).astype(jnp.bfloat16)
    g1 = (ln_gamma @ w_cat).reshape(1, n_types * out_d)
    c_all = (ln_beta @ w_cat).reshape(1, n_types * out_d) \
        + head_b.reshape(1, n_types * out_d)

    return pl.pallas_call(
        functools.partial(_body, n_types=n_types, out_d=out_d, hidden=hidden,
                          n_split=1),
        out_shape=jax.ShapeDtypeStruct((n, out_d), jnp.float32),
        grid=grid,
        in_specs=[
            pl.BlockSpec((bm, 1), lambda i: (i, 0)),
            pl.BlockSpec((bm, latent), lambda i: (i, 0)),
            pl.BlockSpec((latent, hidden), lambda i: (0, 0)),
            pl.BlockSpec((1, hidden), lambda i: (0, 0)),
            pl.BlockSpec((hidden, hidden), lambda i: (0, 0)),
            pl.BlockSpec((1, hidden), lambda i: (0, 0)),
            pl.BlockSpec((hidden, n_types * out_d), lambda i: (0, 0)),
            pl.BlockSpec((1, n_types * out_d), lambda i: (0, 0)),
            pl.BlockSpec((1, n_types * out_d), lambda i: (0, 0)),
        ],
        out_specs=pl.BlockSpec((bm, out_d), lambda i: (i, 0)),
        compiler_params=pltpu.CompilerParams(
            dimension_semantics=("parallel",),
            vmem_limit_bytes=56 * 1024 * 1024,
        ),
        name="adaptive_decoder",
        interpret=interpret,
    )(t2, node_latent, w1b, b1r, w2b, b2r, wp, g1, c_all)
